# Initial kernel scaffold; baseline (speedup 1.0000x reference)
import jax
import jax.numpy as jnp
from jax.experimental import pallas as pl


def kernel(points1, points2, xyz1, xyz2, W1, b1, g1, be1, W2, b2, g2, be2):
    raise NotImplementedError("write your pallas kernel here")


# speedup vs baseline: 16.1600x; 16.1600x over previous
import functools

import jax
import jax.numpy as jnp
from jax.experimental import pallas as pl


def _dot(a, b):
    return jax.lax.dot_general(a, b, (((1,), (0,)), ((), ())),
                               preferred_element_type=jnp.float32)


def _k1_body(x2_ref, x1_ref, p1_ref, p2_ref, w1a_ref, w1b_ref, b1_ref,
             y1_ref, s_ref, ss_ref):
    x2b = x2_ref[0]
    x1b = x1_ref[0]
    mm = jax.lax.dot_general(x2b, x1b, (((0,), (0,)), ((), ())),
                             preferred_element_type=jnp.float32)
    x2sq = jnp.sum(x2b * x2b, axis=0)[:, None]
    x1sq = jnp.sum(x1b * x1b, axis=0)[None, :]
    d = x2sq + x1sq - 2.0 * mm
    m1 = jnp.min(d, axis=1, keepdims=True)
    d2 = jnp.where(d > m1, d, jnp.inf)
    m2 = jnp.min(d2, axis=1, keepdims=True)
    d3 = jnp.where(d2 > m2, d2, jnp.inf)
    m3 = jnp.min(d3, axis=1, keepdims=True)
    w1v = 1.0 / jnp.maximum(jnp.maximum(m1, 0.0), 1e-16)
    w2v = 1.0 / jnp.maximum(jnp.maximum(m2, 0.0), 1e-16)
    w3v = 1.0 / jnp.maximum(jnp.maximum(m3, 0.0), 1e-16)
    tot = w1v + w2v + w3v
    wm = jnp.where(d == m1, w1v / tot,
                   jnp.where(d == m2, w2v / tot,
                             jnp.where(d == m3, w3v / tot, 0.0)))
    nf = _dot(wm, p1_ref[0])
    y1 = (_dot(nf, w1a_ref[...]) + _dot(p2_ref[0], w1b_ref[...])
          + b1_ref[...])

    @pl.when(jnp.logical_and(pl.program_id(0) == 0, pl.program_id(1) == 0))
    def _():
        s_ref[...] = jnp.zeros_like(s_ref)
        ss_ref[...] = jnp.zeros_like(ss_ref)

    y1_ref[0] = y1
    s_ref[...] += jnp.sum(y1, axis=0, keepdims=True)
    ss_ref[...] += jnp.sum(y1 * y1, axis=0, keepdims=True)


def _k2_body(y1_ref, a1_ref, c1_ref, w2t_ref, b2_ref, y2_ref, s_ref, ss_ref):
    z = jnp.maximum(y1_ref[0] * a1_ref[...] + c1_ref[...], 0.0)
    y2 = _dot(z, w2t_ref[...]) + b2_ref[...]

    @pl.when(jnp.logical_and(pl.program_id(0) == 0, pl.program_id(1) == 0))
    def _():
        s_ref[...] = jnp.zeros_like(s_ref)
        ss_ref[...] = jnp.zeros_like(ss_ref)

    y2_ref[0] = y2
    s_ref[...] += jnp.sum(y2, axis=0, keepdims=True)
    ss_ref[...] += jnp.sum(y2 * y2, axis=0, keepdims=True)


def _k3_body(y2_ref, a2_ref, c2_ref, o_ref):
    o_ref[0] = jnp.maximum(y2_ref[0] * a2_ref[...] + c2_ref[...], 0.0)


def _forward(points1, points2, xyz1, xyz2, W1, b1, g1, be1, W2, b2, g2, be2,
             interpret=False):
    B, S, C = points1.shape
    N = points2.shape[1]
    H1 = W1.shape[0]
    H2 = W2.shape[0]
    Nb = min(256, N)
    nblk = N // Nb

    x1t = jnp.transpose(xyz1, (0, 2, 1))
    x2t = jnp.transpose(xyz2, (0, 2, 1))
    w1aT = jnp.transpose(W1[:, :C])
    w1bT = jnp.transpose(W1[:, C:])
    w2T = jnp.transpose(W2)
    b1r = b1.reshape(1, H1)
    b2r = b2.reshape(1, H2)

    y1, s1, ss1 = pl.pallas_call(
        _k1_body,
        grid=(B, nblk),
        in_specs=[
            pl.BlockSpec((1, 3, Nb), lambda b, n: (b, 0, n)),
            pl.BlockSpec((1, 3, S), lambda b, n: (b, 0, 0)),
            pl.BlockSpec((1, S, C), lambda b, n: (b, 0, 0)),
            pl.BlockSpec((1, Nb, C), lambda b, n: (b, n, 0)),
            pl.BlockSpec((C, H1), lambda b, n: (0, 0)),
            pl.BlockSpec((C, H1), lambda b, n: (0, 0)),
            pl.BlockSpec((1, H1), lambda b, n: (0, 0)),
        ],
        out_specs=[
            pl.BlockSpec((1, Nb, H1), lambda b, n: (b, n, 0)),
            pl.BlockSpec((1, H1), lambda b, n: (0, 0)),
            pl.BlockSpec((1, H1), lambda b, n: (0, 0)),
        ],
        out_shape=[
            jax.ShapeDtypeStruct((B, N, H1), jnp.float32),
            jax.ShapeDtypeStruct((1, H1), jnp.float32),
            jax.ShapeDtypeStruct((1, H1), jnp.float32),
        ],
        interpret=interpret,
    )(x2t, x1t, points1, points2, w1aT, w1bT, b1r)

    cnt = float(B * N)
    mean1 = s1[0] / cnt
    var1 = ss1[0] / cnt - mean1 * mean1
    a1 = g1 / jnp.sqrt(var1 + 1e-5)
    c1 = be1 - mean1 * a1

    y2, s2, ss2 = pl.pallas_call(
        _k2_body,
        grid=(B, nblk),
        in_specs=[
            pl.BlockSpec((1, Nb, H1), lambda b, n: (b, n, 0)),
            pl.BlockSpec((1, H1), lambda b, n: (0, 0)),
            pl.BlockSpec((1, H1), lambda b, n: (0, 0)),
            pl.BlockSpec((H1, H2), lambda b, n: (0, 0)),
            pl.BlockSpec((1, H2), lambda b, n: (0, 0)),
        ],
        out_specs=[
            pl.BlockSpec((1, Nb, H2), lambda b, n: (b, n, 0)),
            pl.BlockSpec((1, H2), lambda b, n: (0, 0)),
            pl.BlockSpec((1, H2), lambda b, n: (0, 0)),
        ],
        out_shape=[
            jax.ShapeDtypeStruct((B, N, H2), jnp.float32),
            jax.ShapeDtypeStruct((1, H2), jnp.float32),
            jax.ShapeDtypeStruct((1, H2), jnp.float32),
        ],
        interpret=interpret,
    )(y1, a1.reshape(1, H1), c1.reshape(1, H1), w2T, b2r)

    mean2 = s2[0] / cnt
    var2 = ss2[0] / cnt - mean2 * mean2
    a2 = g2 / jnp.sqrt(var2 + 1e-5)
    c2 = be2 - mean2 * a2

    Nb3 = min(2048, N)
    out = pl.pallas_call(
        _k3_body,
        grid=(B, N // Nb3),
        in_specs=[
            pl.BlockSpec((1, Nb3, H2), lambda b, n: (b, n, 0)),
            pl.BlockSpec((1, H2), lambda b, n: (0, 0)),
            pl.BlockSpec((1, H2), lambda b, n: (0, 0)),
        ],
        out_specs=pl.BlockSpec((1, Nb3, H2), lambda b, n: (b, n, 0)),
        out_shape=jax.ShapeDtypeStruct((B, N, H2), jnp.float32),
        interpret=interpret,
    )(y2, a2.reshape(1, H2), c2.reshape(1, H2))
    return out


def kernel(points1, points2, xyz1, xyz2, W1, b1, g1, be1, W2, b2, g2, be2):
    return _forward(points1, points2, xyz1, xyz2, W1, b1, g1, be1,
                    W2, b2, g2, be2)


# speedup vs baseline: 18.7459x; 1.1600x over previous
import functools

import jax
import jax.numpy as jnp
from jax.experimental import pallas as pl
from jax.experimental.pallas import tpu as pltpu

_CH = 128


def _dot(a, b):
    return jax.lax.dot_general(a, b, (((1,), (0,)), ((), ())),
                               preferred_element_type=jnp.float32)


def _k1_body(x2_ref, x1_ref, p1_ref, p2_ref, w1a_ref, w1b_ref, b1_ref,
             y1_ref, s_ref, ss_ref, d_ref):
    x2b = x2_ref[0]
    x1b = x1_ref[0]
    nb = x2b.shape[1]
    s_len = x1b.shape[1]
    mm = jax.lax.dot_general(-2.0 * x2b, x1b, (((0,), (0,)), ((), ())),
                             preferred_element_type=jnp.float32)
    x2sq = jnp.sum(x2b * x2b, axis=0)[:, None]
    x1sq = jnp.sum(x1b * x1b, axis=0)[None, :]
    t1 = jnp.full((nb, _CH), jnp.inf, jnp.float32)
    t2 = t1
    t3 = t1
    for c in range(s_len // _CH):
        lo = c * _CH
        hi = lo + _CH
        d_c = (x2sq + x1sq[:, lo:hi]) + mm[:, lo:hi]
        d_ref[:, lo:hi] = d_c
        a = jnp.minimum(t1, d_c)
        b = jnp.maximum(t1, d_c)
        t1 = a
        a2 = jnp.minimum(t2, b)
        b2 = jnp.maximum(t2, b)
        t2 = a2
        t3 = jnp.minimum(t3, b2)
    cat = jnp.concatenate([t1, t2, t3], axis=1)
    m1 = jnp.min(cat, axis=1, keepdims=True)
    c2 = jnp.where(cat > m1, cat, jnp.inf)
    m2 = jnp.min(c2, axis=1, keepdims=True)
    c3 = jnp.where(c2 > m2, c2, jnp.inf)
    m3 = jnp.min(c3, axis=1, keepdims=True)
    w1v = 1.0 / jnp.maximum(jnp.maximum(m1, 0.0), 1e-16)
    w2v = 1.0 / jnp.maximum(jnp.maximum(m2, 0.0), 1e-16)
    w3v = 1.0 / jnp.maximum(jnp.maximum(m3, 0.0), 1e-16)
    tot = w1v + w2v + w3v
    d = d_ref[...]
    wm = jnp.where(d == m1, w1v / tot,
                   jnp.where(d == m2, w2v / tot,
                             jnp.where(d == m3, w3v / tot, 0.0)))
    nf = _dot(wm, p1_ref[0])
    y1 = (_dot(nf, w1a_ref[...]) + _dot(p2_ref[0], w1b_ref[...])
          + b1_ref[...])

    @pl.when(jnp.logical_and(pl.program_id(0) == 0, pl.program_id(1) == 0))
    def _():
        s_ref[...] = jnp.zeros_like(s_ref)
        ss_ref[...] = jnp.zeros_like(ss_ref)

    y1_ref[0] = y1
    s_ref[...] += jnp.sum(y1, axis=0, keepdims=True)
    ss_ref[...] += jnp.sum(y1 * y1, axis=0, keepdims=True)


def _k2_body(y1_ref, a1_ref, c1_ref, w2t_ref, b2_ref, y2_ref, s_ref, ss_ref):
    z = jnp.maximum(y1_ref[0] * a1_ref[...] + c1_ref[...], 0.0)
    y2 = _dot(z, w2t_ref[...]) + b2_ref[...]

    @pl.when(jnp.logical_and(pl.program_id(0) == 0, pl.program_id(1) == 0))
    def _():
        s_ref[...] = jnp.zeros_like(s_ref)
        ss_ref[...] = jnp.zeros_like(ss_ref)

    y2_ref[0] = y2
    s_ref[...] += jnp.sum(y2, axis=0, keepdims=True)
    ss_ref[...] += jnp.sum(y2 * y2, axis=0, keepdims=True)


def _k3_body(y2_ref, a2_ref, c2_ref, o_ref):
    o_ref[0] = jnp.maximum(y2_ref[0] * a2_ref[...] + c2_ref[...], 0.0)


def _forward(points1, points2, xyz1, xyz2, W1, b1, g1, be1, W2, b2, g2, be2,
             interpret=False):
    B, S, C = points1.shape
    N = points2.shape[1]
    H1 = W1.shape[0]
    H2 = W2.shape[0]
    Nb = min(256, N)
    nblk = N // Nb

    x1t = jnp.transpose(xyz1, (0, 2, 1))
    x2t = jnp.transpose(xyz2, (0, 2, 1))
    w1aT = jnp.transpose(W1[:, :C])
    w1bT = jnp.transpose(W1[:, C:])
    w2T = jnp.transpose(W2)
    b1r = b1.reshape(1, H1)
    b2r = b2.reshape(1, H2)

    y1, s1, ss1 = pl.pallas_call(
        _k1_body,
        grid=(B, nblk),
        in_specs=[
            pl.BlockSpec((1, 3, Nb), lambda b, n: (b, 0, n)),
            pl.BlockSpec((1, 3, S), lambda b, n: (b, 0, 0)),
            pl.BlockSpec((1, S, C), lambda b, n: (b, 0, 0)),
            pl.BlockSpec((1, Nb, C), lambda b, n: (b, n, 0)),
            pl.BlockSpec((C, H1), lambda b, n: (0, 0)),
            pl.BlockSpec((C, H1), lambda b, n: (0, 0)),
            pl.BlockSpec((1, H1), lambda b, n: (0, 0)),
        ],
        out_specs=[
            pl.BlockSpec((1, Nb, H1), lambda b, n: (b, n, 0)),
            pl.BlockSpec((1, H1), lambda b, n: (0, 0)),
            pl.BlockSpec((1, H1), lambda b, n: (0, 0)),
        ],
        out_shape=[
            jax.ShapeDtypeStruct((B, N, H1), jnp.float32),
            jax.ShapeDtypeStruct((1, H1), jnp.float32),
            jax.ShapeDtypeStruct((1, H1), jnp.float32),
        ],
        scratch_shapes=[pltpu.VMEM((Nb, S), jnp.float32)],
        interpret=interpret,
    )(x2t, x1t, points1, points2, w1aT, w1bT, b1r)

    cnt = float(B * N)
    mean1 = s1[0] / cnt
    var1 = ss1[0] / cnt - mean1 * mean1
    a1 = g1 / jnp.sqrt(var1 + 1e-5)
    c1 = be1 - mean1 * a1

    y2, s2, ss2 = pl.pallas_call(
        _k2_body,
        grid=(B, nblk),
        in_specs=[
            pl.BlockSpec((1, Nb, H1), lambda b, n: (b, n, 0)),
            pl.BlockSpec((1, H1), lambda b, n: (0, 0)),
            pl.BlockSpec((1, H1), lambda b, n: (0, 0)),
            pl.BlockSpec((H1, H2), lambda b, n: (0, 0)),
            pl.BlockSpec((1, H2), lambda b, n: (0, 0)),
        ],
        out_specs=[
            pl.BlockSpec((1, Nb, H2), lambda b, n: (b, n, 0)),
            pl.BlockSpec((1, H2), lambda b, n: (0, 0)),
            pl.BlockSpec((1, H2), lambda b, n: (0, 0)),
        ],
        out_shape=[
            jax.ShapeDtypeStruct((B, N, H2), jnp.float32),
            jax.ShapeDtypeStruct((1, H2), jnp.float32),
            jax.ShapeDtypeStruct((1, H2), jnp.float32),
        ],
        interpret=interpret,
    )(y1, a1.reshape(1, H1), c1.reshape(1, H1), w2T, b2r)

    mean2 = s2[0] / cnt
    var2 = ss2[0] / cnt - mean2 * mean2
    a2 = g2 / jnp.sqrt(var2 + 1e-5)
    c2 = be2 - mean2 * a2

    Nb3 = min(2048, N)
    out = pl.pallas_call(
        _k3_body,
        grid=(B, N // Nb3),
        in_specs=[
            pl.BlockSpec((1, Nb3, H2), lambda b, n: (b, n, 0)),
            pl.BlockSpec((1, H2), lambda b, n: (0, 0)),
            pl.BlockSpec((1, H2), lambda b, n: (0, 0)),
        ],
        out_specs=pl.BlockSpec((1, Nb3, H2), lambda b, n: (b, n, 0)),
        out_shape=jax.ShapeDtypeStruct((B, N, H2), jnp.float32),
        interpret=interpret,
    )(y2, a2.reshape(1, H2), c2.reshape(1, H2))
    return out


def kernel(points1, points2, xyz1, xyz2, W1, b1, g1, be1, W2, b2, g2, be2):
    return _forward(points1, points2, xyz1, xyz2, W1, b1, g1, be1,
                    W2, b2, g2, be2)


# speedup vs baseline: 18.7521x; 1.0003x over previous
import functools

import jax
import jax.numpy as jnp
from jax.experimental import pallas as pl
from jax.experimental.pallas import tpu as pltpu

_CH = 128


def _dot(a, b):
    return jax.lax.dot_general(a, b, (((1,), (0,)), ((), ())),
                               preferred_element_type=jnp.float32)


def _k1_body(x2_ref, x1_ref, p1_ref, p2_ref, w1a_ref, w1b_ref, b1_ref,
             y1_ref, s_ref, ss_ref, d_ref):
    x2b = x2_ref[0]
    x1b = x1_ref[0]
    nb = x2b.shape[1]
    s_len = x1b.shape[1]
    mm = jax.lax.dot_general(-2.0 * x2b, x1b, (((0,), (0,)), ((), ())),
                             preferred_element_type=jnp.float32)
    x2sq = jnp.sum(x2b * x2b, axis=0)[:, None]
    x1sq = jnp.sum(x1b * x1b, axis=0)[None, :]
    RG = 64
    cats = []
    for rg in range(nb // RG):
        r0 = rg * RG
        t1 = jnp.full((RG, _CH), jnp.inf, jnp.float32)
        t2 = t1
        t3 = t1
        x2sq_r = x2sq[r0:r0 + RG]
        for c in range(s_len // _CH):
            lo = c * _CH
            d_c = (x2sq_r + x1sq[:, lo:lo + _CH]) + mm[r0:r0 + RG, lo:lo + _CH]
            d_ref[r0:r0 + RG, lo:lo + _CH] = d_c
            a = jnp.minimum(t1, d_c)
            b = jnp.maximum(t1, d_c)
            t1 = a
            a2 = jnp.minimum(t2, b)
            b2 = jnp.maximum(t2, b)
            t2 = a2
            t3 = jnp.minimum(t3, b2)
        cats.append(jnp.concatenate([t1, t2, t3], axis=1))
    cat = jnp.concatenate(cats, axis=0)
    m1 = jnp.min(cat, axis=1, keepdims=True)
    c2 = jnp.where(cat > m1, cat, jnp.inf)
    m2 = jnp.min(c2, axis=1, keepdims=True)
    c3 = jnp.where(c2 > m2, c2, jnp.inf)
    m3 = jnp.min(c3, axis=1, keepdims=True)
    w1v = 1.0 / jnp.maximum(jnp.maximum(m1, 0.0), 1e-16)
    w2v = 1.0 / jnp.maximum(jnp.maximum(m2, 0.0), 1e-16)
    w3v = 1.0 / jnp.maximum(jnp.maximum(m3, 0.0), 1e-16)
    tot = w1v + w2v + w3v
    d = d_ref[...]
    wm = jnp.where(d == m1, w1v / tot,
                   jnp.where(d == m2, w2v / tot,
                             jnp.where(d == m3, w3v / tot, 0.0)))
    nf = _dot(wm, p1_ref[0])
    y1 = (_dot(nf, w1a_ref[...]) + _dot(p2_ref[0], w1b_ref[...])
          + b1_ref[...])

    @pl.when(jnp.logical_and(pl.program_id(0) == 0, pl.program_id(1) == 0))
    def _():
        s_ref[...] = jnp.zeros_like(s_ref)
        ss_ref[...] = jnp.zeros_like(ss_ref)

    y1_ref[0] = y1
    s_ref[...] += jnp.sum(y1, axis=0, keepdims=True)
    ss_ref[...] += jnp.sum(y1 * y1, axis=0, keepdims=True)


def _k2_body(y1_ref, a1_ref, c1_ref, w2t_ref, b2_ref, y2_ref, s_ref, ss_ref):
    z = jnp.maximum(y1_ref[0] * a1_ref[...] + c1_ref[...], 0.0)
    y2 = _dot(z, w2t_ref[...]) + b2_ref[...]

    @pl.when(jnp.logical_and(pl.program_id(0) == 0, pl.program_id(1) == 0))
    def _():
        s_ref[...] = jnp.zeros_like(s_ref)
        ss_ref[...] = jnp.zeros_like(ss_ref)

    y2_ref[0] = y2
    s_ref[...] += jnp.sum(y2, axis=0, keepdims=True)
    ss_ref[...] += jnp.sum(y2 * y2, axis=0, keepdims=True)


def _k3_body(y2_ref, a2_ref, c2_ref, o_ref):
    o_ref[0] = jnp.maximum(y2_ref[0] * a2_ref[...] + c2_ref[...], 0.0)


def _forward(points1, points2, xyz1, xyz2, W1, b1, g1, be1, W2, b2, g2, be2,
             interpret=False):
    B, S, C = points1.shape
    N = points2.shape[1]
    H1 = W1.shape[0]
    H2 = W2.shape[0]
    Nb = min(256, N)
    nblk = N // Nb

    x1t = jnp.transpose(xyz1, (0, 2, 1))
    x2t = jnp.transpose(xyz2, (0, 2, 1))
    w1aT = jnp.transpose(W1[:, :C])
    w1bT = jnp.transpose(W1[:, C:])
    w2T = jnp.transpose(W2)
    b1r = b1.reshape(1, H1)
    b2r = b2.reshape(1, H2)

    y1, s1, ss1 = pl.pallas_call(
        _k1_body,
        grid=(B, nblk),
        in_specs=[
            pl.BlockSpec((1, 3, Nb), lambda b, n: (b, 0, n)),
            pl.BlockSpec((1, 3, S), lambda b, n: (b, 0, 0)),
            pl.BlockSpec((1, S, C), lambda b, n: (b, 0, 0)),
            pl.BlockSpec((1, Nb, C), lambda b, n: (b, n, 0)),
            pl.BlockSpec((C, H1), lambda b, n: (0, 0)),
            pl.BlockSpec((C, H1), lambda b, n: (0, 0)),
            pl.BlockSpec((1, H1), lambda b, n: (0, 0)),
        ],
        out_specs=[
            pl.BlockSpec((1, Nb, H1), lambda b, n: (b, n, 0)),
            pl.BlockSpec((1, H1), lambda b, n: (0, 0)),
            pl.BlockSpec((1, H1), lambda b, n: (0, 0)),
        ],
        out_shape=[
            jax.ShapeDtypeStruct((B, N, H1), jnp.float32),
            jax.ShapeDtypeStruct((1, H1), jnp.float32),
            jax.ShapeDtypeStruct((1, H1), jnp.float32),
        ],
        scratch_shapes=[pltpu.VMEM((Nb, S), jnp.float32)],
        interpret=interpret,
    )(x2t, x1t, points1, points2, w1aT, w1bT, b1r)

    cnt = float(B * N)
    mean1 = s1[0] / cnt
    var1 = ss1[0] / cnt - mean1 * mean1
    a1 = g1 / jnp.sqrt(var1 + 1e-5)
    c1 = be1 - mean1 * a1

    y2, s2, ss2 = pl.pallas_call(
        _k2_body,
        grid=(B, nblk),
        in_specs=[
            pl.BlockSpec((1, Nb, H1), lambda b, n: (b, n, 0)),
            pl.BlockSpec((1, H1), lambda b, n: (0, 0)),
            pl.BlockSpec((1, H1), lambda b, n: (0, 0)),
            pl.BlockSpec((H1, H2), lambda b, n: (0, 0)),
            pl.BlockSpec((1, H2), lambda b, n: (0, 0)),
        ],
        out_specs=[
            pl.BlockSpec((1, Nb, H2), lambda b, n: (b, n, 0)),
            pl.BlockSpec((1, H2), lambda b, n: (0, 0)),
            pl.BlockSpec((1, H2), lambda b, n: (0, 0)),
        ],
        out_shape=[
            jax.ShapeDtypeStruct((B, N, H2), jnp.float32),
            jax.ShapeDtypeStruct((1, H2), jnp.float32),
            jax.ShapeDtypeStruct((1, H2), jnp.float32),
        ],
        interpret=interpret,
    )(y1, a1.reshape(1, H1), c1.reshape(1, H1), w2T, b2r)

    mean2 = s2[0] / cnt
    var2 = ss2[0] / cnt - mean2 * mean2
    a2 = g2 / jnp.sqrt(var2 + 1e-5)
    c2 = be2 - mean2 * a2

    Nb3 = min(2048, N)
    out = pl.pallas_call(
        _k3_body,
        grid=(B, N // Nb3),
        in_specs=[
            pl.BlockSpec((1, Nb3, H2), lambda b, n: (b, n, 0)),
            pl.BlockSpec((1, H2), lambda b, n: (0, 0)),
            pl.BlockSpec((1, H2), lambda b, n: (0, 0)),
        ],
        out_specs=pl.BlockSpec((1, Nb3, H2), lambda b, n: (b, n, 0)),
        out_shape=jax.ShapeDtypeStruct((B, N, H2), jnp.float32),
        interpret=interpret,
    )(y2, a2.reshape(1, H2), c2.reshape(1, H2))
    return out


def kernel(points1, points2, xyz1, xyz2, W1, b1, g1, be1, W2, b2, g2, be2):
    return _forward(points1, points2, xyz1, xyz2, W1, b1, g1, be1,
                    W2, b2, g2, be2)


# speedup vs baseline: 22.5160x; 1.2007x over previous
import functools

import jax
import jax.numpy as jnp
from jax.experimental import pallas as pl
from jax.experimental.pallas import tpu as pltpu

_CH = 128


def _dot(a, b):
    return jax.lax.dot_general(a, b, (((1,), (0,)), ((), ())),
                               preferred_element_type=jnp.float32)


def _k1_body(x2_ref, x1_ref, p1_ref, p2_ref, w1a_ref, w1b_ref, b1_ref,
             y1_ref, s_ref, ss_ref, d_ref):
    x2b = x2_ref[0]
    x1b = x1_ref[0]
    nb = x2b.shape[1]
    s_len = x1b.shape[1]
    mm = jax.lax.dot_general(-2.0 * x2b, x1b, (((0,), (0,)), ((), ())),
                             preferred_element_type=jnp.float32)
    x2sq = jnp.sum(x2b * x2b, axis=0)[:, None]
    x1sq = jnp.sum(x1b * x1b, axis=0)[None, :]
    RG = 64
    cats = []
    for rg in range(nb // RG):
        r0 = rg * RG
        t1 = jnp.full((RG, _CH), jnp.inf, jnp.float32)
        t2 = t1
        t3 = t1
        x2sq_r = x2sq[r0:r0 + RG]
        for c in range(s_len // _CH):
            lo = c * _CH
            d_c = (x2sq_r + x1sq[:, lo:lo + _CH]) + mm[r0:r0 + RG, lo:lo + _CH]
            d_ref[r0:r0 + RG, lo:lo + _CH] = d_c
            a = jnp.minimum(t1, d_c)
            b = jnp.maximum(t1, d_c)
            t1 = a
            a2 = jnp.minimum(t2, b)
            b2 = jnp.maximum(t2, b)
            t2 = a2
            t3 = jnp.minimum(t3, b2)
        cats.append(jnp.concatenate([t1, t2, t3], axis=1))
    cat = jnp.concatenate(cats, axis=0)
    m1 = jnp.min(cat, axis=1, keepdims=True)
    c2 = jnp.where(cat > m1, cat, jnp.inf)
    m2 = jnp.min(c2, axis=1, keepdims=True)
    c3 = jnp.where(c2 > m2, c2, jnp.inf)
    m3 = jnp.min(c3, axis=1, keepdims=True)
    w1v = 1.0 / jnp.maximum(jnp.maximum(m1, 0.0), 1e-16)
    w2v = 1.0 / jnp.maximum(jnp.maximum(m2, 0.0), 1e-16)
    w3v = 1.0 / jnp.maximum(jnp.maximum(m3, 0.0), 1e-16)
    tot = w1v + w2v + w3v
    d = d_ref[...]
    wm = jnp.where(d == m1, w1v / tot,
                   jnp.where(d == m2, w2v / tot,
                             jnp.where(d == m3, w3v / tot, 0.0)))
    nf = _dot(wm, p1_ref[0])
    y1 = (_dot(nf, w1a_ref[...]) + _dot(p2_ref[0], w1b_ref[...])
          + b1_ref[...])

    @pl.when(jnp.logical_and(pl.program_id(0) == 0, pl.program_id(1) == 0))
    def _():
        s_ref[...] = jnp.zeros_like(s_ref)
        ss_ref[...] = jnp.zeros_like(ss_ref)

    y1_ref[0] = y1
    s_ref[...] += jnp.sum(y1, axis=0, keepdims=True)
    ss_ref[...] += jnp.sum(y1 * y1, axis=0, keepdims=True)


def _k2_body(y1_ref, a1_ref, c1_ref, w2t_ref, b2_ref, y2_ref, s_ref, ss_ref):
    z = jnp.maximum(y1_ref[0] * a1_ref[...] + c1_ref[...], 0.0)
    y2 = _dot(z, w2t_ref[...]) + b2_ref[...]

    @pl.when(jnp.logical_and(pl.program_id(0) == 0, pl.program_id(1) == 0))
    def _():
        s_ref[...] = jnp.zeros_like(s_ref)
        ss_ref[...] = jnp.zeros_like(ss_ref)

    y2_ref[0] = y2
    s_ref[...] += jnp.sum(y2, axis=0, keepdims=True)
    ss_ref[...] += jnp.sum(y2 * y2, axis=0, keepdims=True)


def _k3_body(y2_ref, a2_ref, c2_ref, o_ref):
    o_ref[0] = jnp.maximum(y2_ref[0] * a2_ref[...] + c2_ref[...], 0.0)


def _forward(points1, points2, xyz1, xyz2, W1, b1, g1, be1, W2, b2, g2, be2,
             interpret=False):
    B, S, C = points1.shape
    N = points2.shape[1]
    H1 = W1.shape[0]
    H2 = W2.shape[0]
    Nb = min(512, N)
    nblk = N // Nb

    x1t = jnp.transpose(xyz1, (0, 2, 1))
    x2t = jnp.transpose(xyz2, (0, 2, 1))
    w1aT = jnp.transpose(W1[:, :C])
    w1bT = jnp.transpose(W1[:, C:])
    w2T = jnp.transpose(W2)
    b1r = b1.reshape(1, H1)
    b2r = b2.reshape(1, H2)

    y1, s1, ss1 = pl.pallas_call(
        _k1_body,
        grid=(B, nblk),
        in_specs=[
            pl.BlockSpec((1, 3, Nb), lambda b, n: (b, 0, n)),
            pl.BlockSpec((1, 3, S), lambda b, n: (b, 0, 0)),
            pl.BlockSpec((1, S, C), lambda b, n: (b, 0, 0)),
            pl.BlockSpec((1, Nb, C), lambda b, n: (b, n, 0)),
            pl.BlockSpec((C, H1), lambda b, n: (0, 0)),
            pl.BlockSpec((C, H1), lambda b, n: (0, 0)),
            pl.BlockSpec((1, H1), lambda b, n: (0, 0)),
        ],
        out_specs=[
            pl.BlockSpec((1, Nb, H1), lambda b, n: (b, n, 0)),
            pl.BlockSpec((1, H1), lambda b, n: (0, 0)),
            pl.BlockSpec((1, H1), lambda b, n: (0, 0)),
        ],
        out_shape=[
            jax.ShapeDtypeStruct((B, N, H1), jnp.float32),
            jax.ShapeDtypeStruct((1, H1), jnp.float32),
            jax.ShapeDtypeStruct((1, H1), jnp.float32),
        ],
        scratch_shapes=[pltpu.VMEM((Nb, S), jnp.float32)],
        interpret=interpret,
    )(x2t, x1t, points1, points2, w1aT, w1bT, b1r)

    cnt = float(B * N)
    mean1 = s1[0] / cnt
    var1 = ss1[0] / cnt - mean1 * mean1
    a1 = g1 / jnp.sqrt(var1 + 1e-5)
    c1 = be1 - mean1 * a1

    Nb2 = min(1024, N)
    y2, s2, ss2 = pl.pallas_call(
        _k2_body,
        grid=(B, N // Nb2),
        in_specs=[
            pl.BlockSpec((1, Nb2, H1), lambda b, n: (b, n, 0)),
            pl.BlockSpec((1, H1), lambda b, n: (0, 0)),
            pl.BlockSpec((1, H1), lambda b, n: (0, 0)),
            pl.BlockSpec((H1, H2), lambda b, n: (0, 0)),
            pl.BlockSpec((1, H2), lambda b, n: (0, 0)),
        ],
        out_specs=[
            pl.BlockSpec((1, Nb2, H2), lambda b, n: (b, n, 0)),
            pl.BlockSpec((1, H2), lambda b, n: (0, 0)),
            pl.BlockSpec((1, H2), lambda b, n: (0, 0)),
        ],
        out_shape=[
            jax.ShapeDtypeStruct((B, N, H2), jnp.float32),
            jax.ShapeDtypeStruct((1, H2), jnp.float32),
            jax.ShapeDtypeStruct((1, H2), jnp.float32),
        ],
        interpret=interpret,
    )(y1, a1.reshape(1, H1), c1.reshape(1, H1), w2T, b2r)

    mean2 = s2[0] / cnt
    var2 = ss2[0] / cnt - mean2 * mean2
    a2 = g2 / jnp.sqrt(var2 + 1e-5)
    c2 = be2 - mean2 * a2

    Nb3 = min(2048, N)
    out = pl.pallas_call(
        _k3_body,
        grid=(B, N // Nb3),
        in_specs=[
            pl.BlockSpec((1, Nb3, H2), lambda b, n: (b, n, 0)),
            pl.BlockSpec((1, H2), lambda b, n: (0, 0)),
            pl.BlockSpec((1, H2), lambda b, n: (0, 0)),
        ],
        out_specs=pl.BlockSpec((1, Nb3, H2), lambda b, n: (b, n, 0)),
        out_shape=jax.ShapeDtypeStruct((B, N, H2), jnp.float32),
        interpret=interpret,
    )(y2, a2.reshape(1, H2), c2.reshape(1, H2))
    return out


def kernel(points1, points2, xyz1, xyz2, W1, b1, g1, be1, W2, b2, g2, be2):
    return _forward(points1, points2, xyz1, xyz2, W1, b1, g1, be1,
                    W2, b2, g2, be2)


# speedup vs baseline: 24.3058x; 1.0795x over previous
import functools

import jax
import jax.numpy as jnp
from jax.experimental import pallas as pl
from jax.experimental.pallas import tpu as pltpu

_CH = 128


def _dot(a, b):
    return jax.lax.dot_general(a, b, (((1,), (0,)), ((), ())),
                               preferred_element_type=jnp.float32)


def _k1_body(x2_ref, x1_ref, p1_ref, p2_ref, w1a_ref, w1b_ref, b1_ref,
             y1_ref, s_ref, ss_ref, d_ref):
    x2b = x2_ref[0]
    x1b = x1_ref[0]
    nb = x2b.shape[1]
    s_len = x1b.shape[1]
    mm = jax.lax.dot_general(-2.0 * x2b, x1b, (((0,), (0,)), ((), ())),
                             preferred_element_type=jnp.float32)
    x2sq = jnp.sum(x2b * x2b, axis=0)[:, None]
    x1sq = jnp.sum(x1b * x1b, axis=0)[None, :]
    RG = 64
    cats = []
    for rg in range(nb // RG):
        r0 = rg * RG
        t1 = jnp.full((RG, _CH), jnp.inf, jnp.float32)
        t2 = t1
        t3 = t1
        x2sq_r = x2sq[r0:r0 + RG]
        for c in range(s_len // _CH):
            lo = c * _CH
            d_c = (x2sq_r + x1sq[:, lo:lo + _CH]) + mm[r0:r0 + RG, lo:lo + _CH]
            d_ref[r0:r0 + RG, lo:lo + _CH] = d_c
            a = jnp.minimum(t1, d_c)
            b = jnp.maximum(t1, d_c)
            t1 = a
            a2 = jnp.minimum(t2, b)
            b2 = jnp.maximum(t2, b)
            t2 = a2
            t3 = jnp.minimum(t3, b2)
        cats.append(jnp.concatenate([t1, t2, t3], axis=1))
    cat = jnp.concatenate(cats, axis=0)
    m1 = jnp.min(cat, axis=1, keepdims=True)
    c2 = jnp.where(cat > m1, cat, jnp.inf)
    m2 = jnp.min(c2, axis=1, keepdims=True)
    c3 = jnp.where(c2 > m2, c2, jnp.inf)
    m3 = jnp.min(c3, axis=1, keepdims=True)
    w1v = 1.0 / jnp.maximum(jnp.maximum(m1, 0.0), 1e-16)
    w2v = 1.0 / jnp.maximum(jnp.maximum(m2, 0.0), 1e-16)
    w3v = 1.0 / jnp.maximum(jnp.maximum(m3, 0.0), 1e-16)
    tot = w1v + w2v + w3v
    d = d_ref[...]
    wm = jnp.where(d == m1, w1v / tot,
                   jnp.where(d == m2, w2v / tot,
                             jnp.where(d == m3, w3v / tot, 0.0)))
    nf = _dot(wm, p1_ref[0])
    y1 = (_dot(nf, w1a_ref[...]) + _dot(p2_ref[0], w1b_ref[...])
          + b1_ref[...])

    @pl.when(jnp.logical_and(pl.program_id(0) == 0, pl.program_id(1) == 0))
    def _():
        s_ref[...] = jnp.zeros_like(s_ref)
        ss_ref[...] = jnp.zeros_like(ss_ref)

    y1_ref[0] = y1
    s_ref[...] += jnp.sum(y1, axis=0, keepdims=True)
    ss_ref[...] += jnp.sum(y1 * y1, axis=0, keepdims=True)


def _k2_body(y1_ref, a1_ref, c1_ref, w2t_ref, b2_ref, y2_ref, s_ref, ss_ref):
    z = jnp.maximum(y1_ref[0] * a1_ref[...] + c1_ref[...], 0.0)
    y2 = _dot(z, w2t_ref[...]) + b2_ref[...]

    @pl.when(jnp.logical_and(pl.program_id(0) == 0, pl.program_id(1) == 0))
    def _():
        s_ref[...] = jnp.zeros_like(s_ref)
        ss_ref[...] = jnp.zeros_like(ss_ref)

    y2_ref[0] = y2
    s_ref[...] += jnp.sum(y2, axis=0, keepdims=True)
    ss_ref[...] += jnp.sum(y2 * y2, axis=0, keepdims=True)


def _k3_body(y2_ref, a2_ref, c2_ref, o_ref):
    o_ref[0] = jnp.maximum(y2_ref[0] * a2_ref[...] + c2_ref[...], 0.0)


def _forward(points1, points2, xyz1, xyz2, W1, b1, g1, be1, W2, b2, g2, be2,
             interpret=False):
    B, S, C = points1.shape
    N = points2.shape[1]
    H1 = W1.shape[0]
    H2 = W2.shape[0]
    Nb = min(1024, N)
    nblk = N // Nb

    x1t = jnp.transpose(xyz1, (0, 2, 1))
    x2t = jnp.transpose(xyz2, (0, 2, 1))
    w1aT = jnp.transpose(W1[:, :C])
    w1bT = jnp.transpose(W1[:, C:])
    w2T = jnp.transpose(W2)
    b1r = b1.reshape(1, H1)
    b2r = b2.reshape(1, H2)

    y1, s1, ss1 = pl.pallas_call(
        _k1_body,
        grid=(B, nblk),
        in_specs=[
            pl.BlockSpec((1, 3, Nb), lambda b, n: (b, 0, n)),
            pl.BlockSpec((1, 3, S), lambda b, n: (b, 0, 0)),
            pl.BlockSpec((1, S, C), lambda b, n: (b, 0, 0)),
            pl.BlockSpec((1, Nb, C), lambda b, n: (b, n, 0)),
            pl.BlockSpec((C, H1), lambda b, n: (0, 0)),
            pl.BlockSpec((C, H1), lambda b, n: (0, 0)),
            pl.BlockSpec((1, H1), lambda b, n: (0, 0)),
        ],
        out_specs=[
            pl.BlockSpec((1, Nb, H1), lambda b, n: (b, n, 0)),
            pl.BlockSpec((1, H1), lambda b, n: (0, 0)),
            pl.BlockSpec((1, H1), lambda b, n: (0, 0)),
        ],
        out_shape=[
            jax.ShapeDtypeStruct((B, N, H1), jnp.float32),
            jax.ShapeDtypeStruct((1, H1), jnp.float32),
            jax.ShapeDtypeStruct((1, H1), jnp.float32),
        ],
        scratch_shapes=[pltpu.VMEM((Nb, S), jnp.float32)],
        interpret=interpret,
    )(x2t, x1t, points1, points2, w1aT, w1bT, b1r)

    cnt = float(B * N)
    mean1 = s1[0] / cnt
    var1 = ss1[0] / cnt - mean1 * mean1
    a1 = g1 / jnp.sqrt(var1 + 1e-5)
    c1 = be1 - mean1 * a1

    Nb2 = min(2048, N)
    y2, s2, ss2 = pl.pallas_call(
        _k2_body,
        grid=(B, N // Nb2),
        in_specs=[
            pl.BlockSpec((1, Nb2, H1), lambda b, n: (b, n, 0)),
            pl.BlockSpec((1, H1), lambda b, n: (0, 0)),
            pl.BlockSpec((1, H1), lambda b, n: (0, 0)),
            pl.BlockSpec((H1, H2), lambda b, n: (0, 0)),
            pl.BlockSpec((1, H2), lambda b, n: (0, 0)),
        ],
        out_specs=[
            pl.BlockSpec((1, Nb2, H2), lambda b, n: (b, n, 0)),
            pl.BlockSpec((1, H2), lambda b, n: (0, 0)),
            pl.BlockSpec((1, H2), lambda b, n: (0, 0)),
        ],
        out_shape=[
            jax.ShapeDtypeStruct((B, N, H2), jnp.float32),
            jax.ShapeDtypeStruct((1, H2), jnp.float32),
            jax.ShapeDtypeStruct((1, H2), jnp.float32),
        ],
        interpret=interpret,
    )(y1, a1.reshape(1, H1), c1.reshape(1, H1), w2T, b2r)

    mean2 = s2[0] / cnt
    var2 = ss2[0] / cnt - mean2 * mean2
    a2 = g2 / jnp.sqrt(var2 + 1e-5)
    c2 = be2 - mean2 * a2

    Nb3 = min(2048, N)
    out = pl.pallas_call(
        _k3_body,
        grid=(B, N // Nb3),
        in_specs=[
            pl.BlockSpec((1, Nb3, H2), lambda b, n: (b, n, 0)),
            pl.BlockSpec((1, H2), lambda b, n: (0, 0)),
            pl.BlockSpec((1, H2), lambda b, n: (0, 0)),
        ],
        out_specs=pl.BlockSpec((1, Nb3, H2), lambda b, n: (b, n, 0)),
        out_shape=jax.ShapeDtypeStruct((B, N, H2), jnp.float32),
        interpret=interpret,
    )(y2, a2.reshape(1, H2), c2.reshape(1, H2))
    return out


def kernel(points1, points2, xyz1, xyz2, W1, b1, g1, be1, W2, b2, g2, be2):
    return _forward(points1, points2, xyz1, xyz2, W1, b1, g1, be1,
                    W2, b2, g2, be2)
